# baseline (speedup 1.0000x reference)
import jax
import jax.numpy as jnp
from jax.experimental import pallas as pl


def _layer(h_in, src, dst, W, a_s, a_d, b):
    n = h_in.shape[0]
    h = h_in @ W
    s = h @ a_s
    d = h @ a_d
    e = jax.nn.leaky_relu(s[src] + d[dst], negative_slope=0.2)
    m = jax.ops.segment_max(e, dst, num_segments=n)
    m = jnp.where(jnp.isfinite(m), m, 0.0)
    ex = jnp.exp(e - m[dst])
    denom = jax.ops.segment_sum(ex, dst, num_segments=n)
    alpha = ex / (denom[dst] + 1e-16)
    out = jax.ops.segment_sum(h[src] * alpha[:, None], dst, num_segments=n)
    return out + b


def kernel(x, edge_index, W1, a1s, a1d, b1, W2, a2s, a2d, b2, W3, a3s, a3d, b3):
    src = edge_index[0]
    dst = edge_index[1]
    h = jax.nn.relu(_layer(x, src, dst, W1, a1s, a1d, b1))
    h = jax.nn.relu(_layer(h, src, dst, W2, a2s, a2d, b2))
    h = _layer(h, src, dst, W3, a3s, a3d, b3)
    return jax.nn.log_softmax(h, axis=1)


# speedup vs baseline: 3.1848x; 3.1848x over previous
import functools

import jax
import jax.numpy as jnp
from jax import lax
from jax.experimental import pallas as pl
from jax.experimental.pallas import tpu as pltpu
from jax.experimental.pallas import tpu_sc as plsc

N = 10000
E = 320000
NPAD = 10240
RNG = NPAD // 32
SUB_A = 2016
SUB_B = 4032
CHUNK_A = 10080
EPAD = 32 * CHUNK_A
K = 96
FLUSH_AT = K - 16
F32 = jnp.float32
I32 = jnp.int32
NEG = 0.2


def _lrelu(v):
    return jnp.where(v >= 0, v, NEG * v)


def _mesh():
    return plsc.VectorSubcoreMesh(core_axis_name="c", subcore_axis_name="s")


def _tc_layer(in_dims, d_out, n_out_parts, with_act):
    pw = d_out // n_out_parts

    def body(*refs):
        i = 0
        xs = []
        for _ in in_dims:
            xs.append(refs[i][...])
            i += 1
        if with_act:
            b = refs[i][...]
            i += 1
        W = refs[i][...]
        avs = refs[i + 1][...]
        avd = refs[i + 2][...]
        outs = refs[i + 3:]
        xin = xs[0] if len(xs) == 1 else jnp.concatenate(xs, axis=1)
        if with_act:
            xin = jnp.maximum(xin + b, 0.0)
        h = jnp.dot(xin, W, preferred_element_type=F32)
        for p in range(n_out_parts):
            outs[p][...] = h[:, p * pw:(p + 1) * pw]
        outs[n_out_parts][...] = jnp.dot(h, avs, preferred_element_type=F32)
        outs[n_out_parts + 1][...] = jnp.dot(h, avd, preferred_element_type=F32)

    out_shape = ([jax.ShapeDtypeStruct((NPAD, pw), F32)] * n_out_parts
                 + [jax.ShapeDtypeStruct((NPAD, 1), F32)] * 2)
    return pl.pallas_call(body, out_shape=out_shape)


def _tc_logsoftmax():
    def body(o_ref, b_ref, y_ref):
        xx = o_ref[...][:, :16] + b_ref[...]
        mx = jnp.max(xx, axis=1, keepdims=True)
        ex = jnp.exp(xx - mx)
        lse = jnp.log(jnp.sum(ex, axis=1, keepdims=True)) + mx
        y_ref[...] = xx - lse

    return pl.pallas_call(body, out_shape=jax.ShapeDtypeStruct((NPAD, 16), F32))


def _smax_splat(s_v, sh_v):
    def mstep(i, acc):
        return jnp.maximum(acc, s_v[pl.ds(i * 16, 16)])
    acc = lax.fori_loop(0, NPAD // 16, mstep, jnp.full((16,), -3.4e38, F32))
    idx = lax.iota(I32, 16)
    for sh in (1, 2, 4, 8):
        sh_v[pl.ds(0, 16)] = acc
        acc = jnp.maximum(acc, plsc.load_gather(
            sh_v, [lax.rem(idx + sh, jnp.full((16,), 16, I32))]))
    return acc


def _make_call_a():
    @functools.partial(
        pl.kernel,
        out_type=(jax.ShapeDtypeStruct((32 * NPAD,), F32),
                  jax.ShapeDtypeStruct((EPAD,), F32)),
        mesh=_mesh(),
        compiler_params=pltpu.CompilerParams(needs_layout_passes=False),
        scratch_types=[
            pltpu.VMEM((NPAD,), F32),
            pltpu.VMEM((NPAD,), F32),
            pltpu.VMEM((NPAD,), F32),
            pltpu.VMEM((SUB_A,), I32),
            pltpu.VMEM((SUB_A,), I32),
            pltpu.VMEM((SUB_A,), F32),
            pltpu.VMEM((128,), F32),
        ],
    )
    def call_a(s_hbm, d_hbm, src_hbm, dst_hbm, den_out, ex_out,
               s_v, d_v, den_v, src_v, dst_v, ex_v, sh_v):
        cid = lax.axis_index("c")
        sid = lax.axis_index("s")
        wid = sid * 2 + cid
        pltpu.sync_copy(s_hbm, s_v)
        pltpu.sync_copy(d_hbm, d_v)
        zf = jnp.zeros((16,), F32)

        def zstep(i, c):
            den_v[pl.ds(i * 16, 16)] = zf
            return c
        lax.fori_loop(0, NPAD // 16, zstep, 0)

        smax = _smax_splat(s_v, sh_v)
        base = wid * CHUNK_A

        def sub(j, c):
            off = base + j * SUB_A
            pltpu.sync_copy(src_hbm.at[pl.ds(off, SUB_A)], src_v)
            pltpu.sync_copy(dst_hbm.at[pl.ds(off, SUB_A)], dst_v)

            def step(k2, c2):
                sidx = src_v[pl.ds(k2 * 16, 16)]
                didx = dst_v[pl.ds(k2 * 16, 16)]
                sv = plsc.load_gather(s_v, [sidx])
                dv = plsc.load_gather(d_v, [didx])
                ex = jnp.exp(_lrelu(sv + dv) - _lrelu(smax + dv))
                ex_v[pl.ds(k2 * 16, 16)] = ex
                plsc.addupdate_scatter(den_v, [didx], ex)
                return c2
            lax.fori_loop(0, SUB_A // 16, step, 0)
            pltpu.sync_copy(ex_v, ex_out.at[pl.ds(off, SUB_A)])
            return c
        lax.fori_loop(0, CHUNK_A // SUB_A, sub, 0)
        pltpu.sync_copy(den_v, den_out.at[pl.ds(wid * NPAD, NPAD)])

    return call_a


def _make_call_b(DCH):
    @functools.partial(
        pl.kernel,
        out_type=jax.ShapeDtypeStruct((NPAD, DCH), F32),
        mesh=_mesh(),
        compiler_params=pltpu.CompilerParams(needs_layout_passes=False),
        scratch_types=[
            pltpu.VMEM((RNG, DCH), F32),
            pltpu.VMEM((RNG,), F32),
            pltpu.VMEM((RNG,), F32),
            pltpu.VMEM((SUB_B,), I32),
            pltpu.VMEM((SUB_B,), I32),
            pltpu.VMEM((SUB_B,), F32),
            pltpu.VMEM((K,), I32),
            pltpu.VMEM((K,), I32),
            pltpu.VMEM((K,), F32),
            pltpu.VMEM((K, DCH), F32),
            pltpu.SemaphoreType.DMA,
        ],
    )
    def call_b(src_hbm, dst_hbm, ex_hbm, denp_hbm, h_hbm, out_hbm,
               acc_v, den_v, tmp_v, src_v, dst_v, ex_v,
               gsrc_v, glid_v, galf_v, rows_v, sem):
        cid = lax.axis_index("c")
        sid = lax.axis_index("s")
        rid = cid * 16 + sid
        lo = rid * RNG
        zf = jnp.zeros((16,), F32)
        zi = jnp.zeros((16,), I32)

        pltpu.sync_copy(denp_hbm.at[pl.ds(lo, RNG)], den_v)

        def psum(w, c):
            pltpu.sync_copy(denp_hbm.at[pl.ds(w * NPAD + lo, RNG)], tmp_v)

            def add16(i, c2):
                den_v[pl.ds(i * 16, 16)] = (den_v[pl.ds(i * 16, 16)]
                                            + tmp_v[pl.ds(i * 16, 16)])
                return c2
            lax.fori_loop(0, RNG // 16, add16, 0)
            return c
        lax.fori_loop(1, 32, psum, 0)

        def zacc(i, c):
            for j in range(DCH // 16):
                acc_v[i, pl.ds(j * 16, 16)] = zf
            return c
        lax.fori_loop(0, RNG, zacc, 0)

        def zk(i, c):
            gsrc_v[pl.ds(i * 16, 16)] = zi
            glid_v[pl.ds(i * 16, 16)] = zi
            galf_v[pl.ds(i * 16, 16)] = zf
            return c
        lax.fori_loop(0, K // 16, zk, 0)

        def flush():
            pltpu.async_copy(h_hbm.at[gsrc_v], rows_v, sem).wait()

            def wrow(i, c):
                a = plsc.load_gather(galf_v, [zi + i])
                li = plsc.load_gather(glid_v, [zi + i])[0]
                for j in range(DCH // 16):
                    acc_v[li, pl.ds(j * 16, 16)] = (
                        acc_v[li, pl.ds(j * 16, 16)]
                        + rows_v[i, pl.ds(j * 16, 16)] * a)
                return c
            lax.fori_loop(0, K, wrow, 0)
            lax.fori_loop(0, K // 16, zk, 0)

        def sub(j, cnt):
            off = j * SUB_B
            pltpu.sync_copy(src_hbm.at[pl.ds(off, SUB_B)], src_v)
            pltpu.sync_copy(dst_hbm.at[pl.ds(off, SUB_B)], dst_v)
            pltpu.sync_copy(ex_hbm.at[pl.ds(off, SUB_B)], ex_v)

            def step(k2, cnt2):
                sidx = src_v[pl.ds(k2 * 16, 16)]
                didx = dst_v[pl.ds(k2 * 16, 16)]
                ex = ex_v[pl.ds(k2 * 16, 16)]
                lid = didx - lo
                inr = (lid >= 0) & (lid < RNG)
                lidc = jnp.where(inr, lid, 0)
                den = plsc.load_gather(den_v, [lidc])
                alpha = jnp.where(inr, ex / (den + 1e-16), 0.0)
                plsc.store_compressed(gsrc_v.at[pl.ds(cnt2, 16)], sidx,
                                      mask=inr)
                plsc.store_compressed(glid_v.at[pl.ds(cnt2, 16)], lidc,
                                      mask=inr)
                plsc.store_compressed(galf_v.at[pl.ds(cnt2, 16)], alpha,
                                      mask=inr)
                cnt3 = cnt2 + jnp.sum(inr.astype(I32))
                do = cnt3 > FLUSH_AT
                pl.when(do)(flush)
                return jnp.where(do, 0, cnt3)
            return lax.fori_loop(0, SUB_B // 16, step, cnt)

        cnt = lax.fori_loop(0, EPAD // SUB_B, sub, 0)
        pl.when(cnt > 0)(flush)
        pltpu.sync_copy(acc_v, out_hbm.at[pl.ds(lo, RNG)])

    return call_b


def kernel(x, edge_index, W1, a1s, a1d, b1, W2, a2s, a2d, b2, W3, a3s, a3d, b3):
    xp = jnp.pad(x, ((0, NPAD - N), (0, 0)))
    src = jnp.pad(edge_index[0], (0, EPAD - E), constant_values=NPAD - 1)
    dst = jnp.pad(edge_index[1], (0, EPAD - E), constant_values=NPAD - 1)

    call_a = _make_call_a()
    call_b256 = _make_call_b(256)
    call_b128 = _make_call_b(128)

    h1a, h1b, s1, d1 = _tc_layer([128], 512, 2, False)(
        xp, W1, a1s[:, None], a1d[:, None])
    s1 = s1[:, 0]
    d1 = d1[:, 0]
    denp1, ex1 = call_a(s1, d1, src, dst)
    o1a = call_b256(src, dst, ex1, denp1, h1a)
    o1b = call_b256(src, dst, ex1, denp1, h1b)

    h2, s2, d2 = _tc_layer([256, 256], 256, 1, True)(
        o1a, o1b, b1[None, :], W2, a2s[:, None], a2d[:, None])
    s2 = s2[:, 0]
    d2 = d2[:, 0]
    denp2, ex2 = call_a(s2, d2, src, dst)
    o2 = call_b256(src, dst, ex2, denp2, h2)

    W3p = jnp.pad(W3, ((0, 0), (0, 112)))
    a3sp = jnp.pad(a3s, (0, 112))
    a3dp = jnp.pad(a3d, (0, 112))
    h3, s3, d3 = _tc_layer([256], 128, 1, True)(
        o2, b2[None, :], W3p, a3sp[:, None], a3dp[:, None])
    s3 = s3[:, 0]
    d3 = d3[:, 0]
    denp3, ex3 = call_a(s3, d3, src, dst)
    o3 = call_b128(src, dst, ex3, denp3, h3)

    y = _tc_logsoftmax()(o3, b3[None, :])
    return y[:N]


# speedup vs baseline: 3.8468x; 1.2079x over previous
import functools

import jax
import jax.numpy as jnp
from jax import lax
from jax.experimental import pallas as pl
from jax.experimental.pallas import tpu as pltpu
from jax.experimental.pallas import tpu_sc as plsc

N = 10000
E = 320000
NPAD = 10240
RNG = NPAD // 32
SUB_A = 2048
SUB_B = 2048
CHUNK_A = 10240
EPAD = 32 * CHUNK_A
BLK = 8
K = 128
FLUSH_AT = K - 16
F32 = jnp.float32
I32 = jnp.int32
NEG = 0.2


def _lrelu(v):
    return jnp.where(v >= 0, v, NEG * v)


def _mesh():
    return plsc.VectorSubcoreMesh(core_axis_name="c", subcore_axis_name="s")


def _tc_layer(in_dims, d_out, n_out_parts, with_act):
    pw = d_out // n_out_parts

    def body(*refs):
        i = 0
        xs = []
        for _ in in_dims:
            xs.append(refs[i][...])
            i += 1
        if with_act:
            b = refs[i][...]
            i += 1
        W = refs[i][...]
        avs = refs[i + 1][...]
        avd = refs[i + 2][...]
        outs = refs[i + 3:]
        xin = xs[0] if len(xs) == 1 else jnp.concatenate(xs, axis=1)
        if with_act:
            xin = jnp.maximum(xin + b, 0.0)
        h = jnp.dot(xin, W, preferred_element_type=F32)
        for p in range(n_out_parts):
            outs[p][...] = h[:, p * pw:(p + 1) * pw]
        outs[n_out_parts][...] = jnp.dot(h, avs, preferred_element_type=F32)
        outs[n_out_parts + 1][...] = jnp.dot(h, avd, preferred_element_type=F32)

    out_shape = ([jax.ShapeDtypeStruct((NPAD, pw), F32)] * n_out_parts
                 + [jax.ShapeDtypeStruct((NPAD, 1), F32)] * 2)
    return pl.pallas_call(body, out_shape=out_shape)


def _tc_logsoftmax():
    def body(o_ref, b_ref, y_ref):
        xx = o_ref[...][:, :16] + b_ref[...]
        mx = jnp.max(xx, axis=1, keepdims=True)
        ex = jnp.exp(xx - mx)
        lse = jnp.log(jnp.sum(ex, axis=1, keepdims=True)) + mx
        y_ref[...] = xx - lse

    return pl.pallas_call(body, out_shape=jax.ShapeDtypeStruct((NPAD, 16), F32))


def _smax_splat(s_v, sh_v):
    def mstep(i, acc):
        return jnp.maximum(acc, s_v[pl.ds(i * 16, 16)])
    acc = lax.fori_loop(0, NPAD // 16, mstep, jnp.full((16,), -3.4e38, F32))
    idx = lax.iota(I32, 16)
    for sh in (1, 2, 4, 8):
        sh_v[pl.ds(0, 16)] = acc
        acc = jnp.maximum(acc, plsc.load_gather(
            sh_v, [lax.rem(idx + sh, jnp.full((16,), 16, I32))]))
    return acc


def _make_call_a():
    @functools.partial(
        pl.kernel,
        out_type=(jax.ShapeDtypeStruct((32 * NPAD,), F32),
                  jax.ShapeDtypeStruct((EPAD,), F32)),
        mesh=_mesh(),
        compiler_params=pltpu.CompilerParams(needs_layout_passes=False),
        scratch_types=[
            pltpu.VMEM((NPAD,), F32),
            pltpu.VMEM((NPAD,), F32),
            pltpu.VMEM((NPAD,), F32),
            pltpu.VMEM((SUB_A,), I32),
            pltpu.VMEM((SUB_A,), I32),
            pltpu.VMEM((SUB_A,), F32),
            pltpu.VMEM((128,), F32),
        ],
    )
    def call_a(s_hbm, d_hbm, src_hbm, dst_hbm, den_out, ex_out,
               s_v, d_v, den_v, src_v, dst_v, ex_v, sh_v):
        cid = lax.axis_index("c")
        sid = lax.axis_index("s")
        wid = sid * 2 + cid
        pltpu.sync_copy(s_hbm, s_v)
        pltpu.sync_copy(d_hbm, d_v)
        zf = jnp.zeros((16,), F32)

        def zstep(i, c):
            den_v[pl.ds(i * 16, 16)] = zf
            return c
        lax.fori_loop(0, NPAD // 16, zstep, 0)

        smax = _smax_splat(s_v, sh_v)
        base = wid * CHUNK_A

        def sub(j, c):
            off = base + j * SUB_A
            pltpu.sync_copy(src_hbm.at[pl.ds(off, SUB_A)], src_v)
            pltpu.sync_copy(dst_hbm.at[pl.ds(off, SUB_A)], dst_v)

            def step(k2, c2):
                sidx = src_v[pl.ds(k2 * 16, 16)]
                didx = dst_v[pl.ds(k2 * 16, 16)]
                sv = plsc.load_gather(s_v, [sidx])
                dv = plsc.load_gather(d_v, [didx])
                ex = jnp.exp(_lrelu(sv + dv) - _lrelu(smax + dv))
                ex_v[pl.ds(k2 * 16, 16)] = ex
                plsc.addupdate_scatter(den_v, [didx], ex)
                return c2
            lax.fori_loop(0, SUB_A // 16, step, 0)
            pltpu.sync_copy(ex_v, ex_out.at[pl.ds(off, SUB_A)])
            return c
        lax.fori_loop(0, CHUNK_A // SUB_A, sub, 0)
        pltpu.sync_copy(den_v, den_out.at[pl.ds(wid * NPAD, NPAD)])

    return call_a


def _make_call_b(DCH):
    @functools.partial(
        pl.kernel,
        out_type=jax.ShapeDtypeStruct((NPAD, DCH), F32),
        mesh=_mesh(),
        compiler_params=pltpu.CompilerParams(needs_layout_passes=False),
        scratch_types=[
            pltpu.VMEM((RNG, DCH), F32),
            pltpu.VMEM((RNG,), F32),
            pltpu.VMEM((RNG,), F32),
            pltpu.VMEM((SUB_B,), I32),
            pltpu.VMEM((SUB_B,), I32),
            pltpu.VMEM((SUB_B,), F32),
            pltpu.VMEM((K,), I32),
            pltpu.VMEM((K,), I32),
            pltpu.VMEM((K,), F32),
            pltpu.VMEM((K, DCH), F32),
            pltpu.SemaphoreType.DMA,
        ],
    )
    def call_b(src_hbm, dst_hbm, ex_hbm, denp_hbm, h_hbm, out_hbm,
               acc_v, den_v, tmp_v, src_v, dst_v, ex_v,
               gsrc_v, glid_v, gex_v, rows_v, sem):
        cid = lax.axis_index("c")
        sid = lax.axis_index("s")
        rid = cid * 16 + sid
        lo = rid * RNG
        zf = jnp.zeros((16,), F32)
        zi = jnp.zeros((16,), I32)

        pltpu.sync_copy(denp_hbm.at[pl.ds(lo, RNG)], den_v)

        def psum(w, c):
            pltpu.sync_copy(denp_hbm.at[pl.ds(w * NPAD + lo, RNG)], tmp_v)

            def add16(i, c2):
                den_v[pl.ds(i * 16, 16)] = (den_v[pl.ds(i * 16, 16)]
                                            + tmp_v[pl.ds(i * 16, 16)])
                return c2
            lax.fori_loop(0, RNG // 16, add16, 0)
            return c
        lax.fori_loop(1, 32, psum, 0)

        def dinv(i, c):
            den_v[pl.ds(i * 16, 16)] = 1.0 / (den_v[pl.ds(i * 16, 16)] + 1e-16)
            return c
        lax.fori_loop(0, RNG // 16, dinv, 0)

        def zacc(i, c):
            for j in range(DCH // 16):
                acc_v[i, pl.ds(j * 16, 16)] = zf
            return c
        lax.fori_loop(0, RNG, zacc, 0)

        def zk(i, c):
            gsrc_v[pl.ds(i * 16, 16)] = zi
            glid_v[pl.ds(i * 16, 16)] = zi
            gex_v[pl.ds(i * 16, 16)] = zf
            return c
        lax.fori_loop(0, K // 16, zk, 0)

        def flush():
            pltpu.async_copy(h_hbm.at[gsrc_v], rows_v, sem).wait()

            def wrow(i, c):
                lidv = plsc.load_gather(glid_v, [zi + i])
                exv = plsc.load_gather(gex_v, [zi + i])
                dinv = plsc.load_gather(den_v, [lidv])
                a = exv * dinv
                li = lidv[0]
                for j in range(DCH // 16):
                    plsc.addupdate(acc_v.at[li, pl.ds(j * 16, 16)],
                                   rows_v[i, pl.ds(j * 16, 16)] * a)
                return c
            lax.fori_loop(0, K, wrow, 0)
            lax.fori_loop(0, K // 16, zk, 0)

        def sub(j, cnt):
            off = j * SUB_B
            pltpu.sync_copy(src_hbm.at[pl.ds(off, SUB_B)], src_v)
            pltpu.sync_copy(dst_hbm.at[pl.ds(off, SUB_B)], dst_v)
            pltpu.sync_copy(ex_hbm.at[pl.ds(off, SUB_B)], ex_v)

            def block(g, cnt2):
                vals = []
                for s8 in range(BLK):
                    kbase = (g * BLK + s8) * 16
                    sidx = src_v[pl.ds(kbase, 16)]
                    didx = dst_v[pl.ds(kbase, 16)]
                    ex = ex_v[pl.ds(kbase, 16)]
                    lid = didx - lo
                    inr = (lid >= 0) & (lid < RNG)
                    lidc = jnp.where(inr, lid, 0)
                    vals.append((sidx, lidc, ex, inr))
                for sidx, lidc, ex, inr in vals:
                    plsc.store_compressed(gsrc_v.at[pl.ds(cnt2, 16)], sidx,
                                          mask=inr)
                    plsc.store_compressed(glid_v.at[pl.ds(cnt2, 16)], lidc,
                                          mask=inr)
                    plsc.store_compressed(gex_v.at[pl.ds(cnt2, 16)], ex,
                                          mask=inr)
                    cnt3 = cnt2 + plsc.all_reduce_population_count(inr)[0]
                    do = cnt3 > FLUSH_AT
                    pl.when(do)(flush)
                    cnt2 = jnp.where(do, 0, cnt3)
                return cnt2
            return lax.fori_loop(0, SUB_B // (16 * BLK), block, cnt)

        cnt = lax.fori_loop(0, EPAD // SUB_B, sub, 0)
        pl.when(cnt > 0)(flush)
        pltpu.sync_copy(acc_v, out_hbm.at[pl.ds(lo, RNG)])

    return call_b


def kernel(x, edge_index, W1, a1s, a1d, b1, W2, a2s, a2d, b2, W3, a3s, a3d, b3):
    xp = jnp.pad(x, ((0, NPAD - N), (0, 0)))
    src = jnp.pad(edge_index[0], (0, EPAD - E), constant_values=NPAD - 1)
    dst = jnp.pad(edge_index[1], (0, EPAD - E), constant_values=NPAD - 1)

    call_a = _make_call_a()
    call_b256 = _make_call_b(256)
    call_b128 = _make_call_b(128)

    h1a, h1b, s1, d1 = _tc_layer([128], 512, 2, False)(
        xp, W1, a1s[:, None], a1d[:, None])
    s1 = s1[:, 0]
    d1 = d1[:, 0]
    denp1, ex1 = call_a(s1, d1, src, dst)
    o1a = call_b256(src, dst, ex1, denp1, h1a)
    o1b = call_b256(src, dst, ex1, denp1, h1b)

    h2, s2, d2 = _tc_layer([256, 256], 256, 1, True)(
        o1a, o1b, b1[None, :], W2, a2s[:, None], a2d[:, None])
    s2 = s2[:, 0]
    d2 = d2[:, 0]
    denp2, ex2 = call_a(s2, d2, src, dst)
    o2 = call_b256(src, dst, ex2, denp2, h2)

    W3p = jnp.pad(W3, ((0, 0), (0, 112)))
    a3sp = jnp.pad(a3s, (0, 112))
    a3dp = jnp.pad(a3d, (0, 112))
    h3, s3, d3 = _tc_layer([256], 128, 1, True)(
        o2, b2[None, :], W3p, a3sp[:, None], a3dp[:, None])
    s3 = s3[:, 0]
    d3 = d3[:, 0]
    denp3, ex3 = call_a(s3, d3, src, dst)
    o3 = call_b128(src, dst, ex3, denp3, h3)

    y = _tc_logsoftmax()(o3, b3[None, :])
    return y[:N]


# speedup vs baseline: 4.7630x; 1.2382x over previous
import functools

import jax
import jax.numpy as jnp
from jax import lax
from jax.experimental import pallas as pl
from jax.experimental.pallas import tpu as pltpu
from jax.experimental.pallas import tpu_sc as plsc

N = 10000
E = 320000
NPAD = 10240
RNG = NPAD // 32
RNGP = RNG + 16
SUB = 2048
EPAD = 327680
BLK = 8
K2 = 2048
NBMAX = EPAD // (K2 - 16) + 2
CAP = NBMAX * K2
GR = 64
F32 = jnp.float32
I32 = jnp.int32
NEG = 0.2


def _lrelu(v):
    return jnp.where(v >= 0, v, NEG * v)


def _mesh():
    return plsc.VectorSubcoreMesh(core_axis_name="c", subcore_axis_name="s")


def _tc_layer(in_dims, d_out, n_out_parts, with_act):
    pw = d_out // n_out_parts

    def body(*refs):
        i = 0
        xs = []
        for _ in in_dims:
            xs.append(refs[i][...])
            i += 1
        if with_act:
            b = refs[i][...]
            i += 1
        W = refs[i][...]
        avs = refs[i + 1][...]
        avd = refs[i + 2][...]
        outs = refs[i + 3:]
        xin = xs[0] if len(xs) == 1 else jnp.concatenate(xs, axis=1)
        if with_act:
            xin = jnp.maximum(xin + b, 0.0)
        h = jnp.dot(xin, W, preferred_element_type=F32)
        for p in range(n_out_parts):
            outs[p][...] = h[:, p * pw:(p + 1) * pw]
        outs[n_out_parts][...] = jnp.dot(h, avs, preferred_element_type=F32)
        outs[n_out_parts + 1][...] = jnp.dot(h, avd, preferred_element_type=F32)

    out_shape = ([jax.ShapeDtypeStruct((NPAD, pw), F32)] * n_out_parts
                 + [jax.ShapeDtypeStruct((NPAD, 1), F32)] * 2)
    return pl.pallas_call(body, out_shape=out_shape)


def _tc_logsoftmax():
    def body(o_ref, b_ref, y_ref):
        xx = o_ref[...][:, :16] + b_ref[...]
        mx = jnp.max(xx, axis=1, keepdims=True)
        ex = jnp.exp(xx - mx)
        lse = jnp.log(jnp.sum(ex, axis=1, keepdims=True)) + mx
        y_ref[...] = xx - lse

    return pl.pallas_call(body, out_shape=jax.ShapeDtypeStruct((NPAD, 16), F32))


def _smax_splat(s_v, sh_v):
    def mstep(i, acc):
        return jnp.maximum(acc, s_v[pl.ds(i * 16, 16)])
    acc = lax.fori_loop(0, NPAD // 16, mstep, jnp.full((16,), -3.4e38, F32))
    idx = lax.iota(I32, 16)
    for sh in (1, 2, 4, 8):
        sh_v[pl.ds(0, 16)] = acc
        acc = jnp.maximum(acc, plsc.load_gather(
            sh_v, [lax.rem(idx + sh, jnp.full((16,), 16, I32))]))
    return acc


def _make_partition():
    @functools.partial(
        pl.kernel,
        out_type=(jax.ShapeDtypeStruct((32 * CAP,), I32),
                  jax.ShapeDtypeStruct((32 * CAP,), I32),
                  jax.ShapeDtypeStruct((32 * 16,), I32)),
        mesh=_mesh(),
        compiler_params=pltpu.CompilerParams(needs_layout_passes=False),
        scratch_types=[
            pltpu.VMEM((SUB,), I32),
            pltpu.VMEM((SUB,), I32),
            pltpu.VMEM((K2 + 16,), I32),
            pltpu.VMEM((K2 + 16,), I32),
            pltpu.VMEM((16,), I32),
        ],
    )
    def part(src_hbm, dst_hbm, srcb_out, lidb_out, meta_out,
             src_v, dst_v, csrc_v, clid_v, m_v):
        cid = lax.axis_index("c")
        sid = lax.axis_index("s")
        rid = cid * 16 + sid
        lo = rid * RNG
        zi = jnp.zeros((16,), I32)
        padlid = zi + RNG

        def bflush(cnt3, nblk):
            def go():
                csrc_v[pl.ds(cnt3, 16)] = zi
                clid_v[pl.ds(cnt3, 16)] = padlid
                boff = rid * CAP + nblk * K2
                pltpu.sync_copy(csrc_v.at[pl.ds(0, K2)],
                                srcb_out.at[pl.ds(boff, K2)])
                pltpu.sync_copy(clid_v.at[pl.ds(0, K2)],
                                lidb_out.at[pl.ds(boff, K2)])
            return go

        def sub(j, carry):
            off = j * SUB
            pltpu.sync_copy(src_hbm.at[pl.ds(off, SUB)], src_v)
            pltpu.sync_copy(dst_hbm.at[pl.ds(off, SUB)], dst_v)

            def block(g, carry2):
                cnt2, nblk = carry2
                vals = []
                for s8 in range(BLK):
                    kbase = (g * BLK + s8) * 16
                    sidx = src_v[pl.ds(kbase, 16)]
                    didx = dst_v[pl.ds(kbase, 16)]
                    lid = didx - lo
                    inr = (lid >= 0) & (lid < RNG)
                    lidc = jnp.where(inr, lid, 0)
                    vals.append((sidx, lidc, inr))
                for sidx, lidc, inr in vals:
                    plsc.store_compressed(csrc_v.at[pl.ds(cnt2, 16)], sidx,
                                          mask=inr)
                    plsc.store_compressed(clid_v.at[pl.ds(cnt2, 16)], lidc,
                                          mask=inr)
                    cnt3 = cnt2 + plsc.all_reduce_population_count(inr)[0]
                    do = cnt3 > K2 - 16
                    pl.when(do)(bflush(cnt3, nblk))
                    nblk = jnp.where(do, nblk + 1, nblk)
                    cnt2 = jnp.where(do, 0, cnt3)
                return (cnt2, nblk)
            return lax.fori_loop(0, SUB // (16 * BLK), block, carry)

        cnt, nblk = lax.fori_loop(0, EPAD // SUB, sub, (0, 0))
        pl.when(cnt > 0)(bflush(cnt, nblk))
        nblk = jnp.where(cnt > 0, nblk + 1, nblk)
        m_v[pl.ds(0, 16)] = zi + nblk
        pltpu.sync_copy(m_v, meta_out.at[pl.ds(rid * 16, 16)])

    return part


def _make_layer(DCH, n_passes):
    out_type = ([jax.ShapeDtypeStruct((NPAD, DCH), F32)] * n_passes
                + [jax.ShapeDtypeStruct((32 * CAP,), F32)])

    @functools.partial(
        pl.kernel,
        out_type=out_type,
        mesh=_mesh(),
        compiler_params=pltpu.CompilerParams(needs_layout_passes=False),
        scratch_types=[
            pltpu.VMEM((NPAD,), F32),
            pltpu.VMEM((RNGP,), F32),
            pltpu.VMEM((RNGP,), F32),
            pltpu.VMEM((K2,), I32),
            pltpu.VMEM((K2,), I32),
            pltpu.VMEM((K2,), F32),
            pltpu.VMEM((RNGP, DCH), F32),
            pltpu.VMEM((GR, DCH), F32),
            pltpu.VMEM((128,), F32),
            pltpu.VMEM((16,), I32),
            pltpu.SemaphoreType.DMA,
        ],
    )
    def layer(*args):
        s_hbm, d_hbm, srcb_hbm, lidb_hbm, meta_hbm = args[:5]
        h_parts = args[5:5 + n_passes]
        outs = args[5 + n_passes:5 + 2 * n_passes]
        exb_hbm = args[5 + 2 * n_passes]
        (s_v, d_v, den_v, srcb_v, lidb_v, exb_v, acc_v, rows_v, sh_v,
         m_v, sem) = args[6 + 2 * n_passes:]
        cid = lax.axis_index("c")
        sid = lax.axis_index("s")
        rid = cid * 16 + sid
        lo = rid * RNG
        zf = jnp.zeros((16,), F32)
        zi = jnp.zeros((16,), I32)

        pltpu.sync_copy(s_hbm, s_v)
        pltpu.sync_copy(d_hbm.at[pl.ds(lo, RNG)], d_v.at[pl.ds(0, RNG)])
        d_v[pl.ds(RNG, 16)] = zf
        pltpu.sync_copy(meta_hbm.at[pl.ds(rid * 16, 16)], m_v)
        nblk = m_v[pl.ds(0, 16)][0]
        smax = _smax_splat(s_v, sh_v)

        def zden(i, c):
            den_v[pl.ds(i * 16, 16)] = zf
            return c
        lax.fori_loop(0, RNGP // 16, zden, 0)

        def p1(b, c):
            boff = rid * CAP + b * K2
            pltpu.sync_copy(srcb_hbm.at[pl.ds(boff, K2)], srcb_v)
            pltpu.sync_copy(lidb_hbm.at[pl.ds(boff, K2)], lidb_v)

            def step(k2, c2):
                sidx = srcb_v[pl.ds(k2 * 16, 16)]
                lid = lidb_v[pl.ds(k2 * 16, 16)]
                sv = plsc.load_gather(s_v, [sidx])
                dv = plsc.load_gather(d_v, [lid])
                ex = jnp.exp(_lrelu(sv + dv) - _lrelu(smax + dv))
                exb_v[pl.ds(k2 * 16, 16)] = ex
                plsc.addupdate_scatter(den_v, [lid], ex)
                return c2
            lax.fori_loop(0, K2 // 16, step, 0)
            pltpu.sync_copy(exb_v, exb_hbm.at[pl.ds(boff, K2)])
            return c
        lax.fori_loop(0, nblk, p1, 0)

        def dinv(i, c):
            den_v[pl.ds(i * 16, 16)] = 1.0 / (den_v[pl.ds(i * 16, 16)]
                                              + 1e-16)
            return c
        lax.fori_loop(0, RNGP // 16, dinv, 0)
        den_v[pl.ds(RNG, 16)] = zf

        for p in range(n_passes):
            def zacc(i, c):
                for j in range(DCH // 16):
                    acc_v[i, pl.ds(j * 16, 16)] = zf
                return c
            lax.fori_loop(0, RNGP, zacc, 0)

            h_p = h_parts[p]

            def p2(b, c):
                boff = rid * CAP + b * K2
                pltpu.sync_copy(srcb_hbm.at[pl.ds(boff, K2)], srcb_v)
                pltpu.sync_copy(lidb_hbm.at[pl.ds(boff, K2)], lidb_v)
                pltpu.sync_copy(exb_hbm.at[pl.ds(boff, K2)], exb_v)

                def grp(g, c2):
                    gb = g * GR
                    pltpu.async_copy(
                        h_p.at[srcb_v.at[pl.ds(gb, GR)]], rows_v, sem).wait()

                    def wrow(i, c3):
                        lidv = plsc.load_gather(lidb_v, [zi + gb + i])
                        exv = plsc.load_gather(exb_v, [zi + gb + i])
                        divn = plsc.load_gather(den_v, [lidv])
                        a = exv * divn
                        li = lidv[0]
                        for j in range(DCH // 16):
                            plsc.addupdate(acc_v.at[li, pl.ds(j * 16, 16)],
                                           rows_v[i, pl.ds(j * 16, 16)] * a)
                        return c3
                    lax.fori_loop(0, GR, wrow, 0)
                    return c2
                lax.fori_loop(0, K2 // GR, grp, 0)
                return c
            lax.fori_loop(0, nblk, p2, 0)
            pltpu.sync_copy(acc_v.at[pl.ds(0, RNG)],
                            outs[p].at[pl.ds(lo, RNG)])

    return layer


def kernel(x, edge_index, W1, a1s, a1d, b1, W2, a2s, a2d, b2, W3, a3s, a3d, b3):
    xp = jnp.pad(x, ((0, NPAD - N), (0, 0)))
    src = jnp.pad(edge_index[0], (0, EPAD - E), constant_values=NPAD - 1)
    dst = jnp.pad(edge_index[1], (0, EPAD - E), constant_values=NPAD - 1)

    srcb, lidb, meta = _make_partition()(src, dst)
    layer2p = _make_layer(256, 2)
    layer1p = _make_layer(256, 1)
    layer128 = _make_layer(128, 1)

    h1a, h1b, s1, d1 = _tc_layer([128], 512, 2, False)(
        xp, W1, a1s[:, None], a1d[:, None])
    o1a, o1b, _ = layer2p(s1[:, 0], d1[:, 0], srcb, lidb, meta, h1a, h1b)

    h2, s2, d2 = _tc_layer([256, 256], 256, 1, True)(
        o1a, o1b, b1[None, :], W2, a2s[:, None], a2d[:, None])
    o2, _ = layer1p(s2[:, 0], d2[:, 0], srcb, lidb, meta, h2)

    W3p = jnp.pad(W3, ((0, 0), (0, 112)))
    a3sp = jnp.pad(a3s, (0, 112))
    a3dp = jnp.pad(a3d, (0, 112))
    h3, s3, d3 = _tc_layer([256], 128, 1, True)(
        o2, b2[None, :], W3p, a3sp[:, None], a3dp[:, None])
    o3, _ = layer128(s3[:, 0], d3[:, 0], srcb, lidb, meta, h3)

    y = _tc_logsoftmax()(o3, b3[None, :])
    return y[:N]


# speedup vs baseline: 7.4107x; 1.5559x over previous
import functools

import jax
import jax.numpy as jnp
from jax import lax
from jax.experimental import pallas as pl
from jax.experimental.pallas import tpu as pltpu
from jax.experimental.pallas import tpu_sc as plsc

N = 10000
E = 320000
NPAD = 10240
RNG = NPAD // 32
RNGP = RNG + 16
SUB = 2048
EPAD = 327680
BLK = 8
K2 = 1024
NBMAX = EPAD // (K2 - 16) + 2
CAP = NBMAX * K2
GR = 64
F32 = jnp.float32
I32 = jnp.int32
NEG = 0.2


def _lrelu(v):
    return jnp.where(v >= 0, v, NEG * v)


def _mesh():
    return plsc.VectorSubcoreMesh(core_axis_name="c", subcore_axis_name="s")


def _tc_layer(in_dims, d_out, n_out_parts, with_act):
    pw = d_out // n_out_parts

    def body(*refs):
        i = 0
        xs = []
        for _ in in_dims:
            xs.append(refs[i][...])
            i += 1
        if with_act:
            b = refs[i][...]
            i += 1
        W = refs[i][...]
        avs = refs[i + 1][...]
        avd = refs[i + 2][...]
        outs = refs[i + 3:]
        xin = xs[0] if len(xs) == 1 else jnp.concatenate(xs, axis=1)
        if with_act:
            xin = jnp.maximum(xin + b, 0.0)
        h = jnp.dot(xin, W, preferred_element_type=F32)
        for p in range(n_out_parts):
            outs[p][...] = h[:, p * pw:(p + 1) * pw]
        outs[n_out_parts][...] = jnp.dot(h, avs, preferred_element_type=F32)
        outs[n_out_parts + 1][...] = jnp.dot(h, avd, preferred_element_type=F32)

    out_shape = ([jax.ShapeDtypeStruct((NPAD, pw), F32)] * n_out_parts
                 + [jax.ShapeDtypeStruct((NPAD, 1), F32)] * 2)
    return pl.pallas_call(body, out_shape=out_shape)


def _tc_logsoftmax():
    def body(o_ref, b_ref, y_ref):
        xx = o_ref[...][:, :16] + b_ref[...]
        mx = jnp.max(xx, axis=1, keepdims=True)
        ex = jnp.exp(xx - mx)
        lse = jnp.log(jnp.sum(ex, axis=1, keepdims=True)) + mx
        y_ref[...] = xx - lse

    return pl.pallas_call(body, out_shape=jax.ShapeDtypeStruct((NPAD, 16), F32))


def _smax_splat(s_v, sh_v):
    def mstep(i, acc):
        return jnp.maximum(acc, s_v[pl.ds(i * 16, 16)])
    acc = lax.fori_loop(0, NPAD // 16, mstep, jnp.full((16,), -3.4e38, F32))
    idx = lax.iota(I32, 16)
    for sh in (1, 2, 4, 8):
        sh_v[pl.ds(0, 16)] = acc
        acc = jnp.maximum(acc, plsc.load_gather(
            sh_v, [lax.rem(idx + sh, jnp.full((16,), 16, I32))]))
    return acc


def _make_partition():
    @functools.partial(
        pl.kernel,
        out_type=(jax.ShapeDtypeStruct((32 * CAP,), I32),
                  jax.ShapeDtypeStruct((32 * CAP,), I32),
                  jax.ShapeDtypeStruct((32 * 16,), I32)),
        mesh=_mesh(),
        compiler_params=pltpu.CompilerParams(needs_layout_passes=False),
        scratch_types=[
            pltpu.VMEM((SUB,), I32),
            pltpu.VMEM((SUB,), I32),
            pltpu.VMEM((K2 + 16,), I32),
            pltpu.VMEM((K2 + 16,), I32),
            pltpu.VMEM((16,), I32),
        ],
    )
    def part(src_hbm, dst_hbm, srcb_out, lidb_out, meta_out,
             src_v, dst_v, csrc_v, clid_v, m_v):
        cid = lax.axis_index("c")
        sid = lax.axis_index("s")
        rid = cid * 16 + sid
        lo = rid * RNG
        zi = jnp.zeros((16,), I32)
        padlid = zi + RNG

        def bflush(cnt3, nblk):
            def go():
                csrc_v[pl.ds(cnt3, 16)] = zi
                clid_v[pl.ds(cnt3, 16)] = padlid
                boff = rid * CAP + nblk * K2
                pltpu.sync_copy(csrc_v.at[pl.ds(0, K2)],
                                srcb_out.at[pl.ds(boff, K2)])
                pltpu.sync_copy(clid_v.at[pl.ds(0, K2)],
                                lidb_out.at[pl.ds(boff, K2)])
            return go

        def sub(j, carry):
            off = j * SUB
            pltpu.sync_copy(src_hbm.at[pl.ds(off, SUB)], src_v)
            pltpu.sync_copy(dst_hbm.at[pl.ds(off, SUB)], dst_v)

            def block(g, carry2):
                cnt2, nblk = carry2
                vals = []
                for s8 in range(BLK):
                    kbase = (g * BLK + s8) * 16
                    sidx = src_v[pl.ds(kbase, 16)]
                    didx = dst_v[pl.ds(kbase, 16)]
                    lid = didx - lo
                    inr = (lid >= 0) & (lid < RNG)
                    lidc = jnp.where(inr, lid, 0)
                    vals.append((sidx, lidc, inr))
                for sidx, lidc, inr in vals:
                    plsc.store_compressed(csrc_v.at[pl.ds(cnt2, 16)], sidx,
                                          mask=inr)
                    plsc.store_compressed(clid_v.at[pl.ds(cnt2, 16)], lidc,
                                          mask=inr)
                    cnt3 = cnt2 + plsc.all_reduce_population_count(inr)[0]
                    do = cnt3 > K2 - 16
                    pl.when(do)(bflush(cnt3, nblk))
                    nblk = jnp.where(do, nblk + 1, nblk)
                    cnt2 = jnp.where(do, 0, cnt3)
                return (cnt2, nblk)
            return lax.fori_loop(0, SUB // (16 * BLK), block, carry)

        cnt, nblk = lax.fori_loop(0, EPAD // SUB, sub, (0, 0))
        pl.when(cnt > 0)(bflush(cnt, nblk))
        nblk = jnp.where(cnt > 0, nblk + 1, nblk)
        m_v[pl.ds(0, 16)] = zi + nblk
        pltpu.sync_copy(m_v, meta_out.at[pl.ds(rid * 16, 16)])

    return part


def _make_layer(DCH, n_passes):
    out_type = ([jax.ShapeDtypeStruct((NPAD, DCH), F32)] * n_passes
                + [jax.ShapeDtypeStruct((32 * CAP,), F32)])

    @functools.partial(
        pl.kernel,
        out_type=out_type,
        mesh=_mesh(),
        compiler_params=pltpu.CompilerParams(needs_layout_passes=False),
        scratch_types=[
            pltpu.VMEM((NPAD,), F32),
            pltpu.VMEM((RNGP,), F32),
            pltpu.VMEM((RNGP,), F32),
            pltpu.VMEM((K2,), I32),
            pltpu.VMEM((K2,), I32),
            pltpu.VMEM((K2,), F32),
            pltpu.VMEM((RNG, DCH), F32),
            pltpu.VMEM((GR, DCH), F32),
            pltpu.VMEM((GR, DCH), F32),
            pltpu.VMEM((128,), F32),
            pltpu.VMEM((16,), I32),
            pltpu.SemaphoreType.DMA,
            pltpu.SemaphoreType.DMA,
        ],
    )
    def layer(*args):
        s_hbm, d_hbm, srcb_hbm, lidb_hbm, meta_hbm = args[:5]
        h_parts = args[5:5 + n_passes]
        outs = args[5 + n_passes:5 + 2 * n_passes]
        exb_hbm = args[5 + 2 * n_passes]
        (s_v, d_v, den_v, srcb_v, lidb_v, exb_v, acc_v, rows0_v, rows1_v,
         sh_v, m_v, sem0, sem1) = args[6 + 2 * n_passes:]
        cid = lax.axis_index("c")
        sid = lax.axis_index("s")
        rid = cid * 16 + sid
        lo = rid * RNG
        zf = jnp.zeros((16,), F32)
        zi = jnp.zeros((16,), I32)

        pltpu.sync_copy(s_hbm, s_v)
        pltpu.sync_copy(d_hbm.at[pl.ds(lo, RNG)], d_v.at[pl.ds(0, RNG)])
        d_v[pl.ds(RNG, 16)] = zf
        pltpu.sync_copy(meta_hbm.at[pl.ds(rid * 16, 16)], m_v)
        nblk = m_v[pl.ds(0, 16)][0]
        smax = _smax_splat(s_v, sh_v)

        def zden(i, c):
            den_v[pl.ds(i * 16, 16)] = zf
            return c
        lax.fori_loop(0, RNGP // 16, zden, 0)

        def p1(b, c):
            boff = rid * CAP + b * K2
            pltpu.sync_copy(srcb_hbm.at[pl.ds(boff, K2)], srcb_v)
            pltpu.sync_copy(lidb_hbm.at[pl.ds(boff, K2)], lidb_v)

            def step(k2, c2):
                sidx = srcb_v[pl.ds(k2 * 16, 16)]
                lid = lidb_v[pl.ds(k2 * 16, 16)]
                sv = plsc.load_gather(s_v, [sidx])
                dv = plsc.load_gather(d_v, [lid])
                ex = jnp.exp(_lrelu(sv + dv) - _lrelu(smax + dv))
                exb_v[pl.ds(k2 * 16, 16)] = ex
                plsc.addupdate_scatter(den_v, [lid], ex)
                return c2
            lax.fori_loop(0, K2 // 16, step, 0)
            pltpu.sync_copy(exb_v, exb_hbm.at[pl.ds(boff, K2)])
            return c
        lax.fori_loop(0, nblk, p1, 0)

        def dinv(i, c):
            den_v[pl.ds(i * 16, 16)] = 1.0 / (den_v[pl.ds(i * 16, 16)]
                                              + 1e-16)
            return c
        lax.fori_loop(0, RNGP // 16, dinv, 0)
        den_v[pl.ds(RNG, 16)] = zf

        NG = K2 // GR

        def process(gb, rows_ref):
            def wrow(i, c3):
                lidv = plsc.load_gather(lidb_v, [zi + gb + i])
                a = plsc.load_gather(exb_v, [zi + gb + i])
                li = jnp.minimum(lidv[0], RNG - 1)
                for j in range(DCH // 16):
                    plsc.addupdate(acc_v.at[li, pl.ds(j * 16, 16)],
                                   rows_ref[i, pl.ds(j * 16, 16)] * a)
                return c3
            lax.fori_loop(0, GR, wrow, 0)

        for p in range(n_passes):
            def zacc(i, c):
                for j in range(DCH // 16):
                    acc_v[i, pl.ds(j * 16, 16)] = zf
                return c
            lax.fori_loop(0, RNG, zacc, 0)

            h_p = h_parts[p]

            def p2(b, c):
                boff = rid * CAP + b * K2
                pltpu.sync_copy(srcb_hbm.at[pl.ds(boff, K2)], srcb_v)
                pltpu.sync_copy(lidb_hbm.at[pl.ds(boff, K2)], lidb_v)
                pltpu.sync_copy(exb_hbm.at[pl.ds(boff, K2)], exb_v)

                def mkalpha(k2, c2):
                    lid = lidb_v[pl.ds(k2 * 16, 16)]
                    exv = exb_v[pl.ds(k2 * 16, 16)]
                    exb_v[pl.ds(k2 * 16, 16)] = exv * plsc.load_gather(
                        den_v, [lid])
                    return c2
                lax.fori_loop(0, K2 // 16, mkalpha, 0)

                pltpu.async_copy(
                    h_p.at[srcb_v.at[pl.ds(0, GR)]], rows0_v, sem0)

                def gpair(gp, c2):
                    g0 = gp * 2
                    pltpu.async_copy(
                        h_p.at[srcb_v.at[pl.ds((g0 + 1) * GR, GR)]],
                        rows1_v, sem1)
                    pltpu.make_async_copy(
                        h_p.at[srcb_v.at[pl.ds(g0 * GR, GR)]],
                        rows0_v, sem0).wait()
                    process(g0 * GR, rows0_v)
                    g2 = lax.rem(g0 + 2, NG)
                    pltpu.async_copy(
                        h_p.at[srcb_v.at[pl.ds(g2 * GR, GR)]],
                        rows0_v, sem0)
                    pltpu.make_async_copy(
                        h_p.at[srcb_v.at[pl.ds((g0 + 1) * GR, GR)]],
                        rows1_v, sem1).wait()
                    process((g0 + 1) * GR, rows1_v)
                    return c2
                lax.fori_loop(0, NG // 2, gpair, 0)
                pltpu.make_async_copy(
                    h_p.at[srcb_v.at[pl.ds(0, GR)]], rows0_v, sem0).wait()
                return c
            lax.fori_loop(0, nblk, p2, 0)
            pltpu.sync_copy(acc_v, outs[p].at[pl.ds(lo, RNG)])

    return layer


def kernel(x, edge_index, W1, a1s, a1d, b1, W2, a2s, a2d, b2, W3, a3s, a3d, b3):
    xp = jnp.pad(x, ((0, NPAD - N), (0, 0)))
    src = jnp.pad(edge_index[0], (0, EPAD - E), constant_values=NPAD - 1)
    dst = jnp.pad(edge_index[1], (0, EPAD - E), constant_values=NPAD - 1)

    srcb, lidb, meta = _make_partition()(src, dst)
    layer2p = _make_layer(256, 2)
    layer1p = _make_layer(256, 1)
    layer128 = _make_layer(128, 1)

    h1a, h1b, s1, d1 = _tc_layer([128], 512, 2, False)(
        xp, W1, a1s[:, None], a1d[:, None])
    o1a, o1b, _ = layer2p(s1[:, 0], d1[:, 0], srcb, lidb, meta, h1a, h1b)

    h2, s2, d2 = _tc_layer([256, 256], 256, 1, True)(
        o1a, o1b, b1[None, :], W2, a2s[:, None], a2d[:, None])
    o2, _ = layer1p(s2[:, 0], d2[:, 0], srcb, lidb, meta, h2)

    W3p = jnp.pad(W3, ((0, 0), (0, 112)))
    a3sp = jnp.pad(a3s, (0, 112))
    a3dp = jnp.pad(a3d, (0, 112))
    h3, s3, d3 = _tc_layer([256], 128, 1, True)(
        o2, b2[None, :], W3p, a3sp[:, None], a3dp[:, None])
    o3, _ = layer128(s3[:, 0], d3[:, 0], srcb, lidb, meta, h3)

    y = _tc_logsoftmax()(o3, b3[None, :])
    return y[:N]
